# Initial kernel scaffold; baseline (speedup 1.0000x reference)
import jax
import jax.numpy as jnp
from jax.experimental import pallas as pl


def kernel(xyz, points, W1, b1, g1, be1, W2, b2, g2, be2, Wc1, bc1, Wc2, bc2):
    raise NotImplementedError("write your pallas kernel here")


# speedup vs baseline: 3.5692x; 3.5692x over previous
import functools

import jax
import jax.numpy as jnp
from jax import lax
from jax.experimental import pallas as pl
from jax.experimental.pallas import tpu as pltpu
from jax.experimental.pallas import tpu_sc as plsc

B = 8
N = 4096
S = 1024
K = 16
G = 16
D_FEAT = 32
C0 = 35
C1 = 64
C2 = 64
DP = 48
NW = 32
ROWS_MLP = 131072
BLK_MLP = 2048
SB = 256
NCH = N // 128

_BIG_F = float("inf")
_BIG_I = 2**30


def _fps_body(xs_ref, ys_ref, zs_ref, out_ref):
    X = xs_ref[...]
    Y = ys_ref[...]
    Z = zs_ref[...]
    iota = lax.broadcasted_iota(jnp.int32, (B, N), 1)

    io1024 = lax.broadcasted_iota(jnp.int32, (B, S), 1)
    row = lax.broadcasted_iota(jnp.int32, (B, S), 0)

    def body(i, carry):
        far, dmin, outv = carry
        outv = outv + (io1024 == i).astype(jnp.int32) * far
        onehot = (iota == far).astype(jnp.float32)
        cx = jnp.sum(onehot * X, axis=1, keepdims=True)
        cy = jnp.sum(onehot * Y, axis=1, keepdims=True)
        cz = jnp.sum(onehot * Z, axis=1, keepdims=True)
        dx = X - cx
        dy = Y - cy
        dz = Z - cz
        d = (dx * dx + dy * dy) + dz * dz
        dmin = jnp.minimum(dmin, d)
        m = jnp.max(dmin, axis=1, keepdims=True)
        far2 = jnp.min(iota + (dmin != m).astype(jnp.int32) * _BIG_I,
                       axis=1, keepdims=True)
        return far2.astype(jnp.int32), dmin, outv

    far0 = (xs_ref[:, 0:1] * 0.0).astype(jnp.int32)
    dmin0 = X * 0.0 + 1e10
    outv0 = (io1024 + row) * 0
    _, _, outv = lax.fori_loop(0, S, body, (far0, dmin0, outv0))
    out_ref[...] = outv + row * N


def _fps(xs, ys, zs):
    return pl.pallas_call(
        _fps_body,
        out_shape=jax.ShapeDtypeStruct((B, S), jnp.int32),
    )(xs, ys, zs)


def _make_sc_gather(btot):
    rows = btot // NW
    nch = rows // 128
    mesh = plsc.VectorSubcoreMesh(core_axis_name="c", subcore_axis_name="s")

    @functools.partial(
        pl.kernel,
        mesh=mesh,
        out_type=jax.ShapeDtypeStruct((btot, DP), jnp.float32),
        scratch_types=[
            pltpu.VMEM((128,), jnp.int32),
            pltpu.VMEM((128, DP), jnp.float32),
            pltpu.SemaphoreType.DMA,
        ],
        compiler_params=pltpu.CompilerParams(use_tc_tiling_on_sc=False),
    )
    def gather_kernel(table_hbm, idx_hbm, out_hbm, idx_v, rows_v, sem):
        wid = lax.axis_index("s") * 2 + lax.axis_index("c")
        base = wid * rows

        def body(j, c):
            off = base + j * 128
            pltpu.sync_copy(idx_hbm.at[pl.ds(off, 128)], idx_v)
            pltpu.async_copy(table_hbm.at[idx_v], rows_v, sem).wait()
            pltpu.sync_copy(rows_v, out_hbm.at[pl.ds(off, 128)])
            return c

        lax.fori_loop(0, nch, body, 0)

    return gather_kernel


def _knn_body(q_ref, kt_ref, out_ref, d_ref):
    b = pl.program_id(0)
    q = q_ref[0]
    q2 = q * q
    qn = (q2[:, 0:1] + q2[:, 1:2]) + q2[:, 2:3]
    qb = q.astype(jnp.bfloat16).astype(jnp.float32)
    lane = lax.broadcasted_iota(jnp.int32, (SB, 128), 1)

    def init_chunk(c, carry):
        off = pl.multiple_of(c * 128, 128)
        ktc = kt_ref[0, :, pl.ds(off, 128)]
        k2 = ktc * ktc
        kn = (k2[0:1, :] + k2[1:2, :]) + k2[2:3, :]
        kb = ktc.astype(jnp.bfloat16).astype(jnp.float32)
        mm = (qb[:, 0:1] * kb[0:1, :] + qb[:, 1:2] * kb[1:2, :]) \
            + qb[:, 2:3] * kb[2:3, :]
        d_ref[:, pl.ds(off, 128)] = (qn + kn) - 2.0 * mm
        return carry

    lax.fori_loop(0, NCH, init_chunk, 0)

    io16 = lax.broadcasted_iota(jnp.int32, (SB, K), 1)

    def select(k, carry):
        am_prev, outv = carry

        def chunk(c, mc):
            m, am = mc
            off = pl.multiple_of(c * 128, 128)
            v = d_ref[:, pl.ds(off, 128)]
            gi = lane + off
            v = v + (gi == am_prev).astype(jnp.float32) * 1e30
            d_ref[:, pl.ds(off, 128)] = v
            mc_ = jnp.min(v, axis=1, keepdims=True)
            amc = jnp.min(gi + (v != mc_).astype(jnp.int32) * _BIG_I,
                          axis=1, keepdims=True)
            am2 = jnp.where(mc_ < m, amc,
                            jnp.where(mc_ == m, jnp.minimum(am, amc), am))
            m2 = jnp.minimum(m, mc_)
            return m2, am2

        m0 = qn * 0.0 + 1e30
        a0 = (qn * 0.0).astype(jnp.int32) + _BIG_I
        _, am = lax.fori_loop(0, NCH, chunk, (m0, a0))
        outv = outv + (io16 == k).astype(jnp.int32) * am
        return am, outv

    am0 = (qn * 0.0).astype(jnp.int32) - 1
    io16r = lax.broadcasted_iota(jnp.int32, (SB, K), 0)
    outv0 = (io16 + io16r) * 0
    _, outv = lax.fori_loop(0, K, select, (am0, outv0))
    out_ref[0] = outv + b * N


def _knn(new_xyz, xyzT):
    return pl.pallas_call(
        _knn_body,
        grid=(B, S // SB),
        in_specs=[
            pl.BlockSpec((1, SB, 3), lambda b, s: (b, s, 0)),
            pl.BlockSpec((1, 3, N), lambda b, s: (b, 0, 0)),
        ],
        out_specs=pl.BlockSpec((1, SB, K), lambda b, s: (b, s, 0)),
        out_shape=jax.ShapeDtypeStruct((B, S, K), jnp.int32),
        scratch_shapes=[pltpu.VMEM((SB, N), jnp.float32)],
    )(new_xyz, xyzT)


def _mlp1_body(g_ref, nx_ref, w_ref, b_ref, h_ref, st_ref):
    x = g_ref[...] - nx_ref[...]
    h = jnp.dot(x, w_ref[...], preferred_element_type=jnp.float32) \
        + b_ref[...]
    h_ref[...] = h
    s = jnp.sum(h, axis=0, keepdims=True)
    q = jnp.sum(h * h, axis=0, keepdims=True)
    upd = jnp.concatenate([s, q, jnp.zeros((6, C1), jnp.float32)], axis=0)

    @pl.when(pl.program_id(0) == 0)
    def _():
        st_ref[...] = jnp.zeros_like(st_ref)

    st_ref[...] += upd


def _mlp1(gb, nxrep, w1t, b1r):
    return pl.pallas_call(
        _mlp1_body,
        grid=(ROWS_MLP // BLK_MLP,),
        in_specs=[
            pl.BlockSpec((BLK_MLP, DP), lambda i: (i, 0)),
            pl.BlockSpec((BLK_MLP, DP), lambda i: (i, 0)),
            pl.BlockSpec((DP, C1), lambda i: (0, 0)),
            pl.BlockSpec((1, C1), lambda i: (0, 0)),
        ],
        out_specs=[
            pl.BlockSpec((BLK_MLP, C1), lambda i: (i, 0)),
            pl.BlockSpec((8, C1), lambda i: (0, 0)),
        ],
        out_shape=[
            jax.ShapeDtypeStruct((ROWS_MLP, C1), jnp.float32),
            jax.ShapeDtypeStruct((8, C1), jnp.float32),
        ],
        compiler_params=pltpu.CompilerParams(
            dimension_semantics=("arbitrary",)),
    )(gb, nxrep, w1t, b1r)


def _bn_scale_shift(st, gamma, beta):
    mean = st[0:1, :] / ROWS_MLP
    var = st[1:2, :] / ROWS_MLP - mean * mean
    inv = gamma / jnp.sqrt(var + 1e-5)
    return inv, beta - mean * inv


def _mlp2_body(h_ref, st_ref, g_ref, be_ref, w_ref, b_ref, h2_ref, st2_ref):
    inv, sh = _bn_scale_shift(st_ref[...], g_ref[...], be_ref[...])
    a = jnp.maximum(h_ref[...] * inv + sh, 0.0)
    h2 = jnp.dot(a, w_ref[...], preferred_element_type=jnp.float32) \
        + b_ref[...]
    h2_ref[...] = h2
    s = jnp.sum(h2, axis=0, keepdims=True)
    q = jnp.sum(h2 * h2, axis=0, keepdims=True)
    upd = jnp.concatenate([s, q, jnp.zeros((6, C2), jnp.float32)], axis=0)

    @pl.when(pl.program_id(0) == 0)
    def _():
        st2_ref[...] = jnp.zeros_like(st2_ref)

    st2_ref[...] += upd


def _mlp2(h1, st1, g1r, be1r, w2t, b2r):
    return pl.pallas_call(
        _mlp2_body,
        grid=(ROWS_MLP // BLK_MLP,),
        in_specs=[
            pl.BlockSpec((BLK_MLP, C1), lambda i: (i, 0)),
            pl.BlockSpec((8, C1), lambda i: (0, 0)),
            pl.BlockSpec((1, C1), lambda i: (0, 0)),
            pl.BlockSpec((1, C1), lambda i: (0, 0)),
            pl.BlockSpec((C1, C2), lambda i: (0, 0)),
            pl.BlockSpec((1, C2), lambda i: (0, 0)),
        ],
        out_specs=[
            pl.BlockSpec((BLK_MLP, C2), lambda i: (i, 0)),
            pl.BlockSpec((8, C2), lambda i: (0, 0)),
        ],
        out_shape=[
            jax.ShapeDtypeStruct((ROWS_MLP, C2), jnp.float32),
            jax.ShapeDtypeStruct((8, C2), jnp.float32),
        ],
        compiler_params=pltpu.CompilerParams(
            dimension_semantics=("arbitrary",)),
    )(h1, st1, g1r, be1r, w2t, b2r)


def _mlp3_body(h_ref, st_ref, g_ref, be_ref, out_ref):
    inv, sh = _bn_scale_shift(st_ref[...], g_ref[...], be_ref[...])
    a = jnp.maximum(h_ref[...] * inv + sh, 0.0)
    a3 = a.reshape(BLK_MLP // K, K, C2)
    m = a3[:, 0, :]
    for k in range(1, K):
        m = jnp.maximum(m, a3[:, k, :])
    out_ref[...] = m


def _mlp3(h2, st2, g2r, be2r):
    return pl.pallas_call(
        _mlp3_body,
        grid=(ROWS_MLP // BLK_MLP,),
        in_specs=[
            pl.BlockSpec((BLK_MLP, C2), lambda i: (i, 0)),
            pl.BlockSpec((8, C2), lambda i: (0, 0)),
            pl.BlockSpec((1, C2), lambda i: (0, 0)),
            pl.BlockSpec((1, C2), lambda i: (0, 0)),
        ],
        out_specs=pl.BlockSpec((BLK_MLP // K, C2), lambda i: (i, 0)),
        out_shape=jax.ShapeDtypeStruct((B * S, C2), jnp.float32),
    )(h2, st2, g2r, be2r)


def _fc_body(x_ref, w1_ref, b1_ref, w2_ref, b2_ref, out_ref):
    a = jnp.maximum(
        jnp.dot(x_ref[...], w1_ref[...], preferred_element_type=jnp.float32)
        + b1_ref[...], 0.0)
    out_ref[...] = jnp.dot(a, w2_ref[...],
                           preferred_element_type=jnp.float32) + b2_ref[...]


def _fc(cls_points, wc1t, bc1r, wc2t, bc2r):
    return pl.pallas_call(
        _fc_body,
        out_shape=jax.ShapeDtypeStruct((B * G, C2), jnp.float32),
    )(cls_points, wc1t, bc1r, wc2t, bc2r)


def _sc_gather(table, idx_flat):
    return _make_sc_gather(idx_flat.shape[0])(table, idx_flat)


def kernel(xyz, points, W1, b1, g1, be1, W2, b2, g2, be2, Wc1, bc1, Wc2, bc2):
    f32 = jnp.float32
    xs = xyz[:, :, 0]
    ys = xyz[:, :, 1]
    zs = xyz[:, :, 2]
    xyzT = jnp.transpose(xyz, (0, 2, 1))
    table = jnp.concatenate(
        [xyz, points, jnp.zeros((B, N, DP - C0), f32)], axis=-1
    ).reshape(B * N, DP)

    fps_idx = _fps(xs, ys, zs)
    ga = _sc_gather(table, fps_idx.reshape(B * S))
    new_xyz = ga[:, :3].reshape(B, S, 3)

    idx = _knn(new_xyz, xyzT)
    gb = _sc_gather(table, idx.reshape(ROWS_MLP))

    nxpad = jnp.concatenate(
        [ga[:, :3], jnp.zeros((B * S, DP - 3), f32)], axis=-1)
    nxrep = jnp.repeat(nxpad, K, axis=0)

    w1t = jnp.pad(W1, ((0, 0), (0, DP - C0))).T
    h1, st1 = _mlp1(gb, nxrep, w1t, b1.reshape(1, C1))
    h2, st2 = _mlp2(h1, st1, g1.reshape(1, C1), be1.reshape(1, C1),
                    W2.T, b2.reshape(1, C2))
    sa = _mlp3(h2, st2, g2.reshape(1, C2), be2.reshape(1, C2))

    c = _fc(points[:, :G].reshape(B * G, D_FEAT), Wc1.T,
            bc1.reshape(1, C1), Wc2.T, bc2.reshape(1, C2))

    xyz_out = jnp.concatenate(
        [new_xyz, xyz[:, :G]], axis=1)
    points_out = jnp.concatenate(
        [sa.reshape(B, S, C2), c.reshape(B, G, C2)], axis=1)
    return (xyz_out, points_out)


# speedup vs baseline: 4.9567x; 1.3887x over previous
import functools

import jax
import jax.numpy as jnp
from jax import lax
from jax.experimental import pallas as pl
from jax.experimental.pallas import tpu as pltpu
from jax.experimental.pallas import tpu_sc as plsc

B = 8
N = 4096
S = 1024
K = 16
G = 16
D_FEAT = 32
C0 = 35
C1 = 64
C2 = 64
DP = 48
NW = 32
ROWS_MLP = 131072
BLK_MLP = 2048
SB = 256
NCH = N // 128

_BIG_F = float("inf")
_BIG_I = 2**30


def _fps_body(xs_ref, ys_ref, zs_ref, out_ref):
    X = xs_ref[...]
    Y = ys_ref[...]
    Z = zs_ref[...]
    iota = lax.broadcasted_iota(jnp.int32, (B, N), 1)

    io1024 = lax.broadcasted_iota(jnp.int32, (B, S), 1)
    row = lax.broadcasted_iota(jnp.int32, (B, S), 0)

    def body(i, carry):
        far, dmin, outv = carry
        outv = outv + (io1024 == i).astype(jnp.int32) * far
        onehot = (iota == far).astype(jnp.float32)
        cx = jnp.sum(onehot * X, axis=1, keepdims=True)
        cy = jnp.sum(onehot * Y, axis=1, keepdims=True)
        cz = jnp.sum(onehot * Z, axis=1, keepdims=True)
        dx = X - cx
        dy = Y - cy
        dz = Z - cz
        d = (dx * dx + dy * dy) + dz * dz
        dmin = jnp.minimum(dmin, d)
        m = jnp.max(dmin, axis=1, keepdims=True)
        far2 = jnp.min(iota + (dmin != m).astype(jnp.int32) * _BIG_I,
                       axis=1, keepdims=True)
        return far2.astype(jnp.int32), dmin, outv

    far0 = (xs_ref[:, 0:1] * 0.0).astype(jnp.int32)
    dmin0 = X * 0.0 + 1e10
    outv0 = (io1024 + row) * 0
    _, _, outv = lax.fori_loop(0, S, body, (far0, dmin0, outv0))
    out_ref[...] = outv + row * N


def _fps(xs, ys, zs):
    return pl.pallas_call(
        _fps_body,
        out_shape=jax.ShapeDtypeStruct((B, S), jnp.int32),
    )(xs, ys, zs)


def _make_sc_gather(btot):
    rows = btot // NW
    nch = rows // 128
    mesh = plsc.VectorSubcoreMesh(core_axis_name="c", subcore_axis_name="s")

    @functools.partial(
        pl.kernel,
        mesh=mesh,
        out_type=jax.ShapeDtypeStruct((btot, DP), jnp.float32),
        scratch_types=[
            pltpu.VMEM((128,), jnp.int32),
            pltpu.VMEM((128, DP), jnp.float32),
            pltpu.SemaphoreType.DMA,
        ],
        compiler_params=pltpu.CompilerParams(use_tc_tiling_on_sc=False),
    )
    def gather_kernel(table_hbm, idx_hbm, out_hbm, idx_v, rows_v, sem):
        wid = lax.axis_index("s") * 2 + lax.axis_index("c")
        base = wid * rows

        def body(j, c):
            off = base + j * 128
            pltpu.sync_copy(idx_hbm.at[pl.ds(off, 128)], idx_v)
            pltpu.async_copy(table_hbm.at[idx_v], rows_v, sem).wait()
            pltpu.sync_copy(rows_v, out_hbm.at[pl.ds(off, 128)])
            return c

        lax.fori_loop(0, nch, body, 0)

    return gather_kernel


def _knn_body(q_ref, kt_ref, out_ref, d_ref):
    b = pl.program_id(0)
    q = q_ref[0]
    q2 = q * q
    qn = (q2[:, 0:1] + q2[:, 1:2]) + q2[:, 2:3]
    qb = q.astype(jnp.bfloat16).astype(jnp.float32)
    lane = lax.broadcasted_iota(jnp.int32, (SB, 128), 1)

    def init_chunk(c, carry):
        off = pl.multiple_of(c * 128, 128)
        ktc = kt_ref[0, :, pl.ds(off, 128)]
        k2 = ktc * ktc
        kn = (k2[0:1, :] + k2[1:2, :]) + k2[2:3, :]
        kb = ktc.astype(jnp.bfloat16).astype(jnp.float32)
        mm = (qb[:, 0:1] * kb[0:1, :] + qb[:, 1:2] * kb[1:2, :]) \
            + qb[:, 2:3] * kb[2:3, :]
        d_ref[:, pl.ds(off, 128)] = (qn + kn) - 2.0 * mm
        return carry

    lax.fori_loop(0, NCH, init_chunk, 0, unroll=8)

    io16 = lax.broadcasted_iota(jnp.int32, (SB, K), 1)

    def select(k, carry):
        am_prev, outv = carry

        def chunk(c, mc):
            m, am = mc
            off = pl.multiple_of(c * 128, 128)
            v = d_ref[:, pl.ds(off, 128)]
            gi = lane + off
            v = v + (gi == am_prev).astype(jnp.float32) * 1e30
            d_ref[:, pl.ds(off, 128)] = v
            mc_ = jnp.min(v, axis=1, keepdims=True)
            amc = jnp.min(gi + (v != mc_).astype(jnp.int32) * _BIG_I,
                          axis=1, keepdims=True)
            am2 = jnp.where(mc_ < m, amc,
                            jnp.where(mc_ == m, jnp.minimum(am, amc), am))
            m2 = jnp.minimum(m, mc_)
            return m2, am2

        m0 = qn * 0.0 + 1e30
        a0 = (qn * 0.0).astype(jnp.int32) + _BIG_I
        _, am = lax.fori_loop(0, NCH, chunk, (m0, a0), unroll=8)
        outv = outv + (io16 == k).astype(jnp.int32) * am
        return am, outv

    am0 = (qn * 0.0).astype(jnp.int32) - 1
    io16r = lax.broadcasted_iota(jnp.int32, (SB, K), 0)
    outv0 = (io16 + io16r) * 0
    _, outv = lax.fori_loop(0, K, select, (am0, outv0))
    out_ref[0] = outv + b * N


def _knn(new_xyz, xyzT):
    return pl.pallas_call(
        _knn_body,
        grid=(B, S // SB),
        in_specs=[
            pl.BlockSpec((1, SB, 3), lambda b, s: (b, s, 0)),
            pl.BlockSpec((1, 3, N), lambda b, s: (b, 0, 0)),
        ],
        out_specs=pl.BlockSpec((1, SB, K), lambda b, s: (b, s, 0)),
        out_shape=jax.ShapeDtypeStruct((B, S, K), jnp.int32),
        scratch_shapes=[pltpu.VMEM((SB, N), jnp.float32)],
    )(new_xyz, xyzT)


def _mlp1_body(g_ref, nx_ref, w_ref, b_ref, h_ref, st_ref):
    x = g_ref[...] - nx_ref[...]
    h = jnp.dot(x, w_ref[...], preferred_element_type=jnp.float32) \
        + b_ref[...]
    h_ref[...] = h
    s = jnp.sum(h, axis=0, keepdims=True)
    q = jnp.sum(h * h, axis=0, keepdims=True)
    upd = jnp.concatenate([s, q, jnp.zeros((6, C1), jnp.float32)], axis=0)

    @pl.when(pl.program_id(0) == 0)
    def _():
        st_ref[...] = jnp.zeros_like(st_ref)

    st_ref[...] += upd


def _mlp1(gb, nxrep, w1t, b1r):
    return pl.pallas_call(
        _mlp1_body,
        grid=(ROWS_MLP // BLK_MLP,),
        in_specs=[
            pl.BlockSpec((BLK_MLP, DP), lambda i: (i, 0)),
            pl.BlockSpec((BLK_MLP, DP), lambda i: (i, 0)),
            pl.BlockSpec((DP, C1), lambda i: (0, 0)),
            pl.BlockSpec((1, C1), lambda i: (0, 0)),
        ],
        out_specs=[
            pl.BlockSpec((BLK_MLP, C1), lambda i: (i, 0)),
            pl.BlockSpec((8, C1), lambda i: (0, 0)),
        ],
        out_shape=[
            jax.ShapeDtypeStruct((ROWS_MLP, C1), jnp.float32),
            jax.ShapeDtypeStruct((8, C1), jnp.float32),
        ],
        compiler_params=pltpu.CompilerParams(
            dimension_semantics=("arbitrary",)),
    )(gb, nxrep, w1t, b1r)


def _bn_scale_shift(st, gamma, beta):
    mean = st[0:1, :] / ROWS_MLP
    var = st[1:2, :] / ROWS_MLP - mean * mean
    inv = gamma / jnp.sqrt(var + 1e-5)
    return inv, beta - mean * inv


def _mlp2_body(h_ref, st_ref, g_ref, be_ref, w_ref, b_ref, h2_ref, st2_ref):
    inv, sh = _bn_scale_shift(st_ref[...], g_ref[...], be_ref[...])
    a = jnp.maximum(h_ref[...] * inv + sh, 0.0)
    h2 = jnp.dot(a, w_ref[...], preferred_element_type=jnp.float32) \
        + b_ref[...]
    h2_ref[...] = h2
    s = jnp.sum(h2, axis=0, keepdims=True)
    q = jnp.sum(h2 * h2, axis=0, keepdims=True)
    upd = jnp.concatenate([s, q, jnp.zeros((6, C2), jnp.float32)], axis=0)

    @pl.when(pl.program_id(0) == 0)
    def _():
        st2_ref[...] = jnp.zeros_like(st2_ref)

    st2_ref[...] += upd


def _mlp2(h1, st1, g1r, be1r, w2t, b2r):
    return pl.pallas_call(
        _mlp2_body,
        grid=(ROWS_MLP // BLK_MLP,),
        in_specs=[
            pl.BlockSpec((BLK_MLP, C1), lambda i: (i, 0)),
            pl.BlockSpec((8, C1), lambda i: (0, 0)),
            pl.BlockSpec((1, C1), lambda i: (0, 0)),
            pl.BlockSpec((1, C1), lambda i: (0, 0)),
            pl.BlockSpec((C1, C2), lambda i: (0, 0)),
            pl.BlockSpec((1, C2), lambda i: (0, 0)),
        ],
        out_specs=[
            pl.BlockSpec((BLK_MLP, C2), lambda i: (i, 0)),
            pl.BlockSpec((8, C2), lambda i: (0, 0)),
        ],
        out_shape=[
            jax.ShapeDtypeStruct((ROWS_MLP, C2), jnp.float32),
            jax.ShapeDtypeStruct((8, C2), jnp.float32),
        ],
        compiler_params=pltpu.CompilerParams(
            dimension_semantics=("arbitrary",)),
    )(h1, st1, g1r, be1r, w2t, b2r)


def _mlp3_body(h_ref, st_ref, g_ref, be_ref, out_ref):
    inv, sh = _bn_scale_shift(st_ref[...], g_ref[...], be_ref[...])
    a = jnp.maximum(h_ref[...] * inv + sh, 0.0)
    a3 = a.reshape(BLK_MLP // K, K, C2)
    m = a3[:, 0, :]
    for k in range(1, K):
        m = jnp.maximum(m, a3[:, k, :])
    out_ref[...] = m


def _mlp3(h2, st2, g2r, be2r):
    return pl.pallas_call(
        _mlp3_body,
        grid=(ROWS_MLP // BLK_MLP,),
        in_specs=[
            pl.BlockSpec((BLK_MLP, C2), lambda i: (i, 0)),
            pl.BlockSpec((8, C2), lambda i: (0, 0)),
            pl.BlockSpec((1, C2), lambda i: (0, 0)),
            pl.BlockSpec((1, C2), lambda i: (0, 0)),
        ],
        out_specs=pl.BlockSpec((BLK_MLP // K, C2), lambda i: (i, 0)),
        out_shape=jax.ShapeDtypeStruct((B * S, C2), jnp.float32),
    )(h2, st2, g2r, be2r)


def _fc_body(x_ref, w1_ref, b1_ref, w2_ref, b2_ref, out_ref):
    a = jnp.maximum(
        jnp.dot(x_ref[...], w1_ref[...], preferred_element_type=jnp.float32)
        + b1_ref[...], 0.0)
    out_ref[...] = jnp.dot(a, w2_ref[...],
                           preferred_element_type=jnp.float32) + b2_ref[...]


def _fc(cls_points, wc1t, bc1r, wc2t, bc2r):
    return pl.pallas_call(
        _fc_body,
        out_shape=jax.ShapeDtypeStruct((B * G, C2), jnp.float32),
    )(cls_points, wc1t, bc1r, wc2t, bc2r)


def _sc_gather(table, idx_flat):
    return _make_sc_gather(idx_flat.shape[0])(table, idx_flat)


def kernel(xyz, points, W1, b1, g1, be1, W2, b2, g2, be2, Wc1, bc1, Wc2, bc2):
    f32 = jnp.float32
    xs = xyz[:, :, 0]
    ys = xyz[:, :, 1]
    zs = xyz[:, :, 2]
    xyzT = jnp.transpose(xyz, (0, 2, 1))
    table = jnp.concatenate(
        [xyz, points, jnp.zeros((B, N, DP - C0), f32)], axis=-1
    ).reshape(B * N, DP)

    fps_idx = _fps(xs, ys, zs)
    ga = _sc_gather(table, fps_idx.reshape(B * S))
    new_xyz = ga[:, :3].reshape(B, S, 3)

    idx = _knn(new_xyz, xyzT)
    gb = _sc_gather(table, idx.reshape(ROWS_MLP))

    nxpad = jnp.concatenate(
        [ga[:, :3], jnp.zeros((B * S, DP - 3), f32)], axis=-1)
    nxrep = jnp.repeat(nxpad, K, axis=0)

    w1t = jnp.pad(W1, ((0, 0), (0, DP - C0))).T
    h1, st1 = _mlp1(gb, nxrep, w1t, b1.reshape(1, C1))
    h2, st2 = _mlp2(h1, st1, g1.reshape(1, C1), be1.reshape(1, C1),
                    W2.T, b2.reshape(1, C2))
    sa = _mlp3(h2, st2, g2.reshape(1, C2), be2.reshape(1, C2))

    c = _fc(points[:, :G].reshape(B * G, D_FEAT), Wc1.T,
            bc1.reshape(1, C1), Wc2.T, bc2.reshape(1, C2))

    xyz_out = jnp.concatenate(
        [new_xyz, xyz[:, :G]], axis=1)
    points_out = jnp.concatenate(
        [sa.reshape(B, S, C2), c.reshape(B, G, C2)], axis=1)
    return (xyz_out, points_out)
